# Initial kernel scaffold; baseline (speedup 1.0000x reference)
import jax
import jax.numpy as jnp
from jax.experimental import pallas as pl


def kernel(source, target):
    raise NotImplementedError("write your pallas kernel here")


# speedup vs baseline: 43.4209x; 43.4209x over previous
import functools

import jax
import jax.numpy as jnp
from jax import lax
from jax.experimental import pallas as pl
from jax.experimental.pallas import tpu as pltpu
from jax.experimental.pallas import tpu_sc as plsc

BINS = 256
LANES = 16
CHUNK = 16384


def _make_sc_hist(n_elems):
    mesh = plsc.VectorSubcoreMesh(core_axis_name="c", subcore_axis_name="s")
    n_workers = mesh.num_cores * mesh.num_subcores
    per_tile = n_elems // n_workers
    n_chunks = per_tile // CHUNK
    n_pairs = n_chunks // 2
    assert per_tile * n_workers == n_elems
    assert n_pairs * 2 * CHUNK == per_tile

    @functools.partial(
        pl.kernel,
        out_type=jax.ShapeDtypeStruct((2, n_workers, BINS), jnp.float32),
        mesh=mesh,
        compiler_params=pltpu.CompilerParams(needs_layout_passes=False),
        scratch_types=[
            pltpu.VMEM((2 * CHUNK,), jnp.float32),
            pltpu.VMEM((LANES * BINS,), jnp.float32),
            pltpu.VMEM((LANES * BINS,), jnp.float32),
            pltpu.VMEM((BINS,), jnp.float32),
            pltpu.SemaphoreType.DMA,
            pltpu.SemaphoreType.DMA,
        ],
    )
    def hist_kernel(src_hbm, tgt_hbm, out_hbm, buf, hist_a, hist_b, red, sem0, sem1):
        wid = lax.axis_index("c") * mesh.num_subcores + lax.axis_index("s")
        base = wid * per_tile
        lane_off = lax.iota(jnp.int32, LANES) * BINS
        ones = jnp.full((LANES,), 1.0, jnp.float32)

        def zero_body(k, carry):
            z = jnp.zeros((LANES,), jnp.float32)
            hist_a[pl.ds(k * LANES, LANES)] = z
            hist_b[pl.ds(k * LANES, LANES)] = z
            return carry

        lax.fori_loop(0, (LANES * BINS) // LANES, zero_body, 0)

        def inner(off, hist):
            U = 8

            def body(i, carry):
                b = off + i * (U * LANES)
                for u in range(U):
                    x = buf[pl.ds(b + u * LANES, LANES)]
                    idx = jnp.minimum((x * float(BINS)).astype(jnp.int32),
                                      BINS - 1) + lane_off
                    plsc.addupdate_scatter(hist, [idx], ones)
                return carry

            lax.fori_loop(0, CHUNK // (U * LANES), body, 0)

        def process(src, hist):
            pltpu.async_copy(src.at[pl.ds(base, CHUNK)],
                             buf.at[pl.ds(0, CHUNK)], sem0)

            def pair(p, carry):
                c1 = 2 * p + 1
                pltpu.async_copy(src.at[pl.ds(base + c1 * CHUNK, CHUNK)],
                                 buf.at[pl.ds(CHUNK, CHUNK)], sem1)
                pltpu.make_async_copy(src.at[pl.ds(base, CHUNK)],
                                      buf.at[pl.ds(0, CHUNK)], sem0).wait()
                inner(0, hist)
                c2 = c1 + 1

                @pl.when(c2 < n_chunks)
                def _():
                    pltpu.async_copy(src.at[pl.ds(base + c2 * CHUNK, CHUNK)],
                                     buf.at[pl.ds(0, CHUNK)], sem0)

                pltpu.make_async_copy(src.at[pl.ds(base, CHUNK)],
                                      buf.at[pl.ds(CHUNK, CHUNK)], sem1).wait()
                inner(CHUNK, hist)
                return carry

            lax.fori_loop(0, n_pairs, pair, 0)

        def flush(hist, t):
            for m in range(BINS // LANES):
                acc = hist[pl.ds(m * LANES, LANES)]
                for j in range(1, LANES):
                    acc = acc + hist[pl.ds(j * BINS + m * LANES, LANES)]
                red[pl.ds(m * LANES, LANES)] = acc
            pltpu.sync_copy(red, out_hbm.at[t, wid])

        process(src_hbm, hist_a)
        flush(hist_a, 0)
        process(tgt_hbm, hist_b)
        flush(hist_b, 1)

    return hist_kernel


def _loss_body(p_ref, o_ref):
    p = p_ref[...]
    s = jnp.sum(p[0], axis=0)
    t = jnp.sum(p[1], axis=0)
    eps = 1e-8
    sn = s / jnp.sum(s) + eps
    tn = t / jnp.sum(t) + eps
    d = sn - tn
    o_ref[...] = (jnp.sum(d * d) * (1.0 / BINS)).reshape(1, 1)


def kernel(source, target):
    n = source.size
    sflat = source.reshape((n,))
    tflat = target.reshape((n,))
    partials = _make_sc_hist(n)(sflat, tflat)
    loss = pl.pallas_call(
        _loss_body,
        out_shape=jax.ShapeDtypeStruct((1, 1), jnp.float32),
    )(partials)
    return loss.reshape(())


# speedup vs baseline: 168.6193x; 3.8834x over previous
import functools

import jax
import jax.numpy as jnp
from jax import lax
from jax.experimental import pallas as pl
from jax.experimental.pallas import tpu as pltpu
from jax.experimental.pallas import tpu_sc as plsc

BINS = 256
LANES = 16
CHUNK = 16384


def _make_sc_hist(n_elems):
    mesh = plsc.VectorSubcoreMesh(core_axis_name="c", subcore_axis_name="s")
    n_workers = mesh.num_cores * mesh.num_subcores
    per_tile = n_elems // n_workers
    n_chunks = per_tile // CHUNK
    n_pairs = n_chunks // 2
    assert per_tile * n_workers == n_elems
    assert n_pairs * 2 * CHUNK == per_tile

    @functools.partial(
        pl.kernel,
        out_type=jax.ShapeDtypeStruct((2, n_workers, BINS, LANES), jnp.float32),
        mesh=mesh,
        compiler_params=pltpu.CompilerParams(needs_layout_passes=False),
        scratch_types=[
            pltpu.VMEM((2 * CHUNK,), jnp.float32),
            pltpu.VMEM((BINS, LANES), jnp.float32),
            pltpu.VMEM((BINS, LANES), jnp.float32),
            pltpu.SemaphoreType.DMA,
            pltpu.SemaphoreType.DMA,
        ],
    )
    def hist_kernel(src_hbm, tgt_hbm, out_hbm, buf, hist_a, hist_b, sem0, sem1):
        wid = lax.axis_index("c") * mesh.num_subcores + lax.axis_index("s")
        base = wid * per_tile
        lane = lax.iota(jnp.int32, LANES)
        ones = jnp.full((LANES,), 1.0, jnp.float32)

        def zero_body(k, carry):
            z = jnp.zeros((LANES,), jnp.float32)
            hist_a[k, :] = z
            hist_b[k, :] = z
            return carry

        lax.fori_loop(0, BINS, zero_body, 0)

        def inner(off, hist):
            @plsc.parallel_loop(0, CHUNK // LANES, unroll=8)
            def body(i):
                x = buf[pl.ds(off + i * LANES, LANES)]
                b = jnp.minimum((x * float(BINS)).astype(jnp.int32), BINS - 1)
                plsc.addupdate_scatter(hist, [b, lane], ones)

        def process(src, hist):
            pltpu.async_copy(src.at[pl.ds(base, CHUNK)],
                             buf.at[pl.ds(0, CHUNK)], sem0)

            def pair(p, carry):
                c1 = 2 * p + 1
                pltpu.async_copy(src.at[pl.ds(base + c1 * CHUNK, CHUNK)],
                                 buf.at[pl.ds(CHUNK, CHUNK)], sem1)
                pltpu.make_async_copy(src.at[pl.ds(base, CHUNK)],
                                      buf.at[pl.ds(0, CHUNK)], sem0).wait()
                inner(0, hist)
                c2 = c1 + 1

                @pl.when(c2 < n_chunks)
                def _():
                    pltpu.async_copy(src.at[pl.ds(base + c2 * CHUNK, CHUNK)],
                                     buf.at[pl.ds(0, CHUNK)], sem0)

                pltpu.make_async_copy(src.at[pl.ds(base, CHUNK)],
                                      buf.at[pl.ds(CHUNK, CHUNK)], sem1).wait()
                inner(CHUNK, hist)
                return carry

            lax.fori_loop(0, n_pairs, pair, 0)

        process(src_hbm, hist_a)
        pltpu.sync_copy(hist_a, out_hbm.at[0, wid])
        process(tgt_hbm, hist_b)
        pltpu.sync_copy(hist_b, out_hbm.at[1, wid])

    return hist_kernel


def _loss_body(p_ref, o_ref):
    p = p_ref[...]
    s = jnp.sum(p[0], axis=(0, 2))
    t = jnp.sum(p[1], axis=(0, 2))
    eps = 1e-8
    sn = s / jnp.sum(s) + eps
    tn = t / jnp.sum(t) + eps
    d = sn - tn
    o_ref[...] = (jnp.sum(d * d) * (1.0 / BINS)).reshape(1, 1)


def kernel(source, target):
    n = source.size
    sflat = source.reshape((n,))
    tflat = target.reshape((n,))
    partials = _make_sc_hist(n)(sflat, tflat)
    loss = pl.pallas_call(
        _loss_body,
        out_shape=jax.ShapeDtypeStruct((1, 1), jnp.float32),
    )(partials)
    return loss.reshape(())


# speedup vs baseline: 192.3164x; 1.1405x over previous
import functools

import jax
import jax.numpy as jnp
from jax import lax
from jax.experimental import pallas as pl
from jax.experimental.pallas import tpu as pltpu
from jax.experimental.pallas import tpu_sc as plsc

BINS = 256
LANES = 16
ROWS = 32
COLS = 512


def _make_sc_hist(n_rows):
    mesh = plsc.VectorSubcoreMesh(core_axis_name="c", subcore_axis_name="s")
    n_workers = mesh.num_cores * mesh.num_subcores
    rows_per_tile = n_rows // n_workers
    n_chunks = rows_per_tile // ROWS
    n_pairs = n_chunks // 2
    assert rows_per_tile * n_workers == n_rows
    assert n_pairs * 2 * ROWS == rows_per_tile
    hist_words = (BINS + 1) * LANES

    @functools.partial(
        pl.kernel,
        out_type=jax.ShapeDtypeStruct((2, n_workers, 8, COLS), jnp.float32),
        mesh=mesh,
        compiler_params=pltpu.CompilerParams(
            needs_layout_passes=False,
            use_tc_tiling_on_sc=True,
        ),
        scratch_types=[
            pltpu.VMEM((2, ROWS, COLS), jnp.float32),
            pltpu.VMEM((hist_words,), jnp.float32),
            pltpu.VMEM((hist_words,), jnp.float32),
            pltpu.VMEM((8, COLS), jnp.float32),
            pltpu.SemaphoreType.DMA,
            pltpu.SemaphoreType.DMA,
        ],
    )
    def hist_kernel(src_hbm, tgt_hbm, out_hbm, buf, hist_a, hist_b, stage,
                    sem0, sem1):
        wid = lax.axis_index("c") * mesh.num_subcores + lax.axis_index("s")
        base = wid * rows_per_tile
        lane = lax.iota(jnp.int32, LANES)
        ones = jnp.full((LANES,), 1.0, jnp.float32)

        def zero_body(k, carry):
            z = jnp.zeros((LANES,), jnp.float32)
            hist_a[pl.ds(k * LANES, LANES)] = z
            hist_b[pl.ds(k * LANES, LANES)] = z
            return carry

        lax.fori_loop(0, BINS + 1, zero_body, 0)

        def inner(half, hist):
            @plsc.parallel_loop(0, ROWS)
            def body(r):
                for c in range(COLS // LANES):
                    x = buf[half, r, pl.ds(c * LANES, LANES)]
                    b = (x * float(BINS)).astype(jnp.int32)
                    addr = lax.shift_left(b, 4) | lane
                    plsc.addupdate_scatter(hist, [addr], ones)

        def process(src, hist):
            pltpu.async_copy(src.at[pl.ds(base, ROWS), :], buf.at[0], sem0)

            def pair(p, carry):
                c1 = 2 * p + 1
                pltpu.async_copy(src.at[pl.ds(base + c1 * ROWS, ROWS), :],
                                 buf.at[1], sem1)
                pltpu.make_async_copy(src.at[pl.ds(base, ROWS), :],
                                      buf.at[0], sem0).wait()
                inner(0, hist)
                c2 = c1 + 1

                @pl.when(c2 < n_chunks)
                def _():
                    pltpu.async_copy(src.at[pl.ds(base + c2 * ROWS, ROWS), :],
                                     buf.at[0], sem0)

                pltpu.make_async_copy(src.at[pl.ds(base, ROWS), :],
                                      buf.at[1], sem1).wait()
                inner(1, hist)
                return carry

            lax.fori_loop(0, n_pairs, pair, 0)

        def flush(hist, t):
            for k in range(BINS):
                acc = hist[pl.ds(k * LANES, LANES)]
                if k == BINS - 1:
                    acc = acc + hist[pl.ds(BINS * LANES, LANES)]
                stage[k // 32, pl.ds((k % 32) * LANES, LANES)] = acc
            pltpu.sync_copy(stage, out_hbm.at[t, wid])

        process(src_hbm, hist_a)
        flush(hist_a, 0)
        process(tgt_hbm, hist_b)
        flush(hist_b, 1)

    return hist_kernel


def _loss_body(p_ref, o_ref):
    p = p_ref[...]
    a0 = jnp.sum(p[0], axis=0)
    a1 = jnp.sum(p[1], axis=0)
    sel = (lax.broadcasted_iota(jnp.int32, (COLS, 32), 0) // LANES ==
           lax.broadcasted_iota(jnp.int32, (COLS, 32), 1)).astype(jnp.float32)
    b0 = jax.lax.dot(a0, sel, preferred_element_type=jnp.float32)
    b1 = jax.lax.dot(a1, sel, preferred_element_type=jnp.float32)
    eps = 1e-8
    sn = b0 / jnp.sum(b0) + eps
    tn = b1 / jnp.sum(b1) + eps
    d = sn - tn
    o_ref[...] = (jnp.sum(d * d) * (1.0 / BINS)).reshape(1, 1)


def kernel(source, target):
    m, r, c = source.shape
    s2d = source.reshape((m * r, c))
    t2d = target.reshape((m * r, c))
    partials = _make_sc_hist(m * r)(s2d, t2d)
    loss = pl.pallas_call(
        _loss_body,
        out_shape=jax.ShapeDtypeStruct((1, 1), jnp.float32),
    )(partials)
    return loss.reshape(())


# speedup vs baseline: 268.2357x; 1.3948x over previous
import functools

import jax
import jax.numpy as jnp
from jax import lax
from jax.experimental import pallas as pl
from jax.experimental.pallas import tpu as pltpu
from jax.experimental.pallas import tpu_sc as plsc

BINS = 256
LANES = 16
ROWS = 64
COLS = 512


def _make_sc_hist(n_rows):
    mesh = plsc.VectorSubcoreMesh(core_axis_name="c", subcore_axis_name="s")
    n_workers = mesh.num_cores * mesh.num_subcores
    rows_per_tile = n_rows // n_workers
    n_chunks = rows_per_tile // ROWS
    n_pairs = n_chunks // 2
    assert rows_per_tile * n_workers == n_rows
    assert n_pairs * 2 * ROWS == rows_per_tile
    hist_words = (BINS + 1) * LANES

    @functools.partial(
        pl.kernel,
        out_type=jax.ShapeDtypeStruct((2, n_workers, 8, COLS), jnp.float32),
        mesh=mesh,
        compiler_params=pltpu.CompilerParams(
            needs_layout_passes=False,
            use_tc_tiling_on_sc=True,
        ),
        scratch_types=[
            pltpu.VMEM((2, ROWS, COLS), jnp.float32),
            pltpu.VMEM((hist_words,), jnp.float32),
            pltpu.VMEM((hist_words,), jnp.float32),
            pltpu.VMEM((8, COLS), jnp.float32),
            pltpu.SemaphoreType.DMA,
            pltpu.SemaphoreType.DMA,
        ],
    )
    def hist_kernel(src_hbm, tgt_hbm, out_hbm, buf, hist_a, hist_b, stage,
                    sem0, sem1):
        wid = lax.axis_index("c") * mesh.num_subcores + lax.axis_index("s")
        base = wid * rows_per_tile
        lane = lax.iota(jnp.int32, LANES)
        ones = jnp.full((LANES,), 1.0, jnp.float32)

        def zero_body(k, carry):
            z = jnp.zeros((LANES,), jnp.float32)
            hist_a[pl.ds(k * LANES, LANES)] = z
            hist_b[pl.ds(k * LANES, LANES)] = z
            return carry

        lax.fori_loop(0, BINS + 1, zero_body, 0)

        def inner(half, hist):
            @plsc.parallel_loop(0, ROWS * 4, unroll=2)
            def body(i):
                r = jnp.right_shift(i, 2)
                q = jnp.bitwise_and(i, 3) * (8 * LANES)
                for u in range(8):
                    x = buf[half, r, pl.ds(q + u * LANES, LANES)]
                    b = (x * float(BINS)).astype(jnp.int32)
                    addr = lax.shift_left(b, 4) | lane
                    plsc.addupdate_scatter(hist, [addr], ones)

        def process(src, hist):
            pltpu.async_copy(src.at[pl.ds(base, ROWS), :], buf.at[0], sem0)

            def pair(p, carry):
                c1 = 2 * p + 1
                pltpu.async_copy(src.at[pl.ds(base + c1 * ROWS, ROWS), :],
                                 buf.at[1], sem1)
                pltpu.make_async_copy(src.at[pl.ds(base, ROWS), :],
                                      buf.at[0], sem0).wait()
                inner(0, hist)
                c2 = c1 + 1

                @pl.when(c2 < n_chunks)
                def _():
                    pltpu.async_copy(src.at[pl.ds(base + c2 * ROWS, ROWS), :],
                                     buf.at[0], sem0)

                pltpu.make_async_copy(src.at[pl.ds(base, ROWS), :],
                                      buf.at[1], sem1).wait()
                inner(1, hist)
                return carry

            lax.fori_loop(0, n_pairs, pair, 0)

        def flush(hist, t):
            for k in range(BINS):
                acc = hist[pl.ds(k * LANES, LANES)]
                if k == BINS - 1:
                    acc = acc + hist[pl.ds(BINS * LANES, LANES)]
                stage[k // 32, pl.ds((k % 32) * LANES, LANES)] = acc
            pltpu.sync_copy(stage, out_hbm.at[t, wid])

        process(src_hbm, hist_a)
        flush(hist_a, 0)
        process(tgt_hbm, hist_b)
        flush(hist_b, 1)

    return hist_kernel


def _loss_body(p_ref, o_ref):
    p = p_ref[...]
    a0 = jnp.sum(p[0], axis=0)
    a1 = jnp.sum(p[1], axis=0)
    sel = (lax.broadcasted_iota(jnp.int32, (COLS, 32), 0) // LANES ==
           lax.broadcasted_iota(jnp.int32, (COLS, 32), 1)).astype(jnp.float32)
    b0 = jax.lax.dot(a0, sel, preferred_element_type=jnp.float32)
    b1 = jax.lax.dot(a1, sel, preferred_element_type=jnp.float32)
    eps = 1e-8
    sn = b0 / jnp.sum(b0) + eps
    tn = b1 / jnp.sum(b1) + eps
    d = sn - tn
    o_ref[...] = (jnp.sum(d * d) * (1.0 / BINS)).reshape(1, 1)


def kernel(source, target):
    m, r, c = source.shape
    s2d = source.reshape((m * r, c))
    t2d = target.reshape((m * r, c))
    partials = _make_sc_hist(m * r)(s2d, t2d)
    loss = pl.pallas_call(
        _loss_body,
        out_shape=jax.ShapeDtypeStruct((1, 1), jnp.float32),
    )(partials)
    return loss.reshape(())
